# Initial kernel scaffold; baseline (speedup 1.0000x reference)
import jax
import jax.numpy as jnp
from jax.experimental import pallas as pl


def kernel(tasks, inputs, pre_ln_g, pre_ln_b, bn_g, bn_b, router_w, router_b, w_down, b_down, w_up, b_up, post_ln_g, post_ln_b):
    raise NotImplementedError("write your pallas kernel here")


# speedup vs baseline: 3.4959x; 3.4959x over previous
import jax
import jax.numpy as jnp
from jax.experimental import pallas as pl
from jax.experimental.pallas import tpu as pltpu

_B, _S, _D = 4, 2048, 1024
_E = 8
_DH = _D // 4
_CHUNK = 512
_NC = _S // _CHUNK
_EPS = 1e-5


def _ln(x, g, b):
    mu = jnp.mean(x, axis=-1, keepdims=True)
    xc = x - mu
    var = jnp.mean(xc * xc, axis=-1, keepdims=True)
    return xc * jax.lax.rsqrt(var + _EPS) * g + b


def _adapter_kernel(x_ref, pre_g_ref, pre_b_ref, bn_g_ref, bn_b_ref,
                    rw_ref, rb_ref, wd_ref, bd_ref, wu_ref, bu_ref,
                    post_g_ref, post_b_ref, out_ref):
    pre_g = pre_g_ref[...]
    pre_b = pre_b_ref[...]

    rin_sum = jnp.zeros((1, _D), jnp.float32)
    for c in range(_NC):
        x = x_ref[0, c * _CHUNK:(c + 1) * _CHUNK, :]
        z = _ln(x, pre_g, pre_b)
        rin_sum = rin_sum + jnp.sum(z, axis=0, keepdims=True)

    rin = rin_sum * (1.0 / _S)
    rin = rin * (1.0 / jnp.sqrt(1.0 + _EPS)) * bn_g_ref[...] + bn_b_ref[...]
    logits = jnp.dot(rin, rw_ref[...], preferred_element_type=jnp.float32)
    logits = logits + rb_ref[...]
    m = jnp.max(logits)
    gate = 1.0 / jnp.sum(jnp.exp(logits - m))
    lane = jax.lax.broadcasted_iota(jnp.int32, (1, _E), 1)
    top1 = jnp.min(jnp.where(logits == m, lane, _E))

    wd = wd_ref[top1]
    bd = bd_ref[top1]
    wu = wu_ref[top1]
    bu = bu_ref[top1]
    post_g = post_g_ref[...]
    post_b = post_b_ref[...]

    for c in range(_NC):
        x = x_ref[0, c * _CHUNK:(c + 1) * _CHUNK, :]
        z = _ln(x, pre_g, pre_b)
        h = jnp.dot(z, wd, preferred_element_type=jnp.float32) + bd
        h = jnp.maximum(h, 0.0)
        up = jnp.dot(h, wu, preferred_element_type=jnp.float32) + bu
        up = up * gate
        out_ref[0, c * _CHUNK:(c + 1) * _CHUNK, :] = _ln(up, post_g, post_b) + x


def kernel(tasks, inputs, pre_ln_g, pre_ln_b, bn_g, bn_b, router_w, router_b,
           w_down, b_down, w_up, b_up, post_ln_g, post_ln_b):
    del tasks
    row = lambda v: v.reshape(1, -1)
    full = lambda a: pl.BlockSpec(a.shape, lambda b: (0,) * a.ndim)

    args = (
        inputs,
        row(pre_ln_g), row(pre_ln_b), row(bn_g), row(bn_b),
        router_w, row(router_b),
        w_down, b_down.reshape(_E, 1, _DH),
        w_up, b_up.reshape(_E, 1, _D),
        row(post_ln_g), row(post_ln_b),
    )
    in_specs = [pl.BlockSpec((1, _S, _D), lambda b: (b, 0, 0))]
    in_specs += [full(a) for a in args[1:]]

    return pl.pallas_call(
        _adapter_kernel,
        grid=(_B,),
        in_specs=in_specs,
        out_specs=pl.BlockSpec((1, _S, _D), lambda b: (b, 0, 0)),
        out_shape=jax.ShapeDtypeStruct((_B, _S, _D), jnp.float32),
    )(*args)
